# Initial kernel scaffold; baseline (speedup 1.0000x reference)
import jax
import jax.numpy as jnp
from jax.experimental import pallas as pl


def kernel(patch_features, E1, E2, temperature, fusion_weight, W1, b1, ln_g, ln_b, W2, b2, We1, be1, We2, be2):
    raise NotImplementedError("write your pallas kernel here")


# speedup vs baseline: 6.2151x; 6.2151x over previous
import functools
import jax
import jax.numpy as jnp
from jax import lax
from jax.experimental import pallas as pl
from jax.experimental.pallas import tpu as pltpu

_B, _N, _P, _D = 4, 1024, 72, 96
_H, _ND, _K = 4, 64, 32


def _kth_mask(y, k):
    bits = lax.bitcast_convert_type(y, jnp.int32)
    rows = y.shape[0]
    thr0 = jnp.zeros((rows, 1), jnp.int32)

    def body(i, thr):
        b = 30 - i
        cand = thr | (jnp.int32(1) << b)
        cnt = jnp.sum((bits >= cand).astype(jnp.float32), axis=-1,
                      keepdims=True)
        return jnp.where(cnt >= k, cand, thr)

    thr = lax.fori_loop(0, 31, body, thr0)
    return bits >= thr


def _topk_row_block(y, k):
    m = jnp.max(y, axis=-1, keepdims=True)
    z = jnp.exp(y - m)
    s_full = jnp.sum(z, axis=-1, keepdims=True)
    mask = _kth_mask(y, k)
    zm = jnp.where(mask, z, 0.0)
    s_top = jnp.sum(zm, axis=-1, keepdims=True)
    return zm / (s_top + 1e-8 * s_full)


def _bf16_dot(a, b):
    return jnp.dot(a.astype(jnp.bfloat16), b.astype(jnp.bfloat16),
                   preferred_element_type=jnp.float32)


def _dyn_emb_body(pf_ref, avg_ref, w1_ref, b1_ref, g_ref, bb_ref, w2_ref,
                  b2_ref, out_ref):
    x = pf_ref[0]
    nr = jnp.dot(x, avg_ref[...], preferred_element_type=jnp.float32, precision=lax.Precision.HIGHEST)
    h = _bf16_dot(nr, w1_ref[...])
    h = h + b1_ref[...]
    mu = jnp.mean(h, axis=-1, keepdims=True)
    var = jnp.mean((h - mu) ** 2, axis=-1, keepdims=True)
    h = (h - mu) / jnp.sqrt(var + 1e-5) * g_ref[...] + bb_ref[...]
    h = jnp.maximum(h, 0.0)
    out_ref[0] = _bf16_dot(h, w2_ref[...]) + b2_ref[...]


def _static_body(e1_ref, e2_ref, t_ref, out_ref):
    t = t_ref[0, 0, 0]
    y = jnp.maximum(_bf16_dot(e1_ref[0], e2_ref[0]), 0.0) / t
    out_ref[0] = _topk_row_block(y, _K)


def _dyn_body(de_ref, e2_ref, t_ref, ss_ref, fw_ref, we1_ref, be1_ref,
              we2_ref, be2_ref, dyn_ref, fin_ref):
    demb = de_ref[0]
    w = fw_ref[0, 0]
    fused = []
    sumf = None
    for h in range(_H):
        y = jnp.maximum(_bf16_dot(demb, e2_ref[h]), 0.0) / t_ref[0, h]
        d = _topk_row_block(y, _K)
        dyn_ref[0, h] = d
        f = (1.0 - w) * ss_ref[h] + w * d
        fused.append(f)
        sumf = f if sumf is None else sumf + f
    ew = be2_ref[0, 0]
    for j in range(2 * _H):
        hid = be1_ref[0, j]
        for h in range(_H):
            hid = hid + fused[h] * we1_ref[h, j]
        hid = jnp.maximum(hid, 0.0)
        ew = ew + hid * we2_ref[0, j]
    sig = 1.0 / (1.0 + jnp.exp(-ew))
    fin_ref[0] = sig * (sumf * (1.0 / _H))


def _run(pf_flat, avg, E1, E2, t_row, w1, b1, g, bb, w2, b2, fw, we1, be1,
         we2_row, be2, blk1=128, blk2=256, blk3=128):
    f32 = jnp.float32

    dyn_emb = pl.pallas_call(
        _dyn_emb_body,
        grid=(_B, _N // blk1),
        in_specs=[
            pl.BlockSpec((1, blk1, _P * _D), lambda b, i: (b, i, 0)),
            pl.BlockSpec((_P * _D, _D), lambda b, i: (0, 0)),
            pl.BlockSpec((_D, _ND), lambda b, i: (0, 0)),
            pl.BlockSpec((1, _ND), lambda b, i: (0, 0)),
            pl.BlockSpec((1, _ND), lambda b, i: (0, 0)),
            pl.BlockSpec((1, _ND), lambda b, i: (0, 0)),
            pl.BlockSpec((_ND, _ND), lambda b, i: (0, 0)),
            pl.BlockSpec((1, _ND), lambda b, i: (0, 0)),
        ],
        out_specs=pl.BlockSpec((1, blk1, _ND), lambda b, i: (b, i, 0)),
        out_shape=jax.ShapeDtypeStruct((_B, _N, _ND), f32),
    )(pf_flat, avg, w1, b1, g, bb, w2, b2)

    static_s = pl.pallas_call(
        _static_body,
        grid=(_H, _N // blk2),
        in_specs=[
            pl.BlockSpec((1, blk2, _ND), lambda h, i: (h, i, 0)),
            pl.BlockSpec((1, _ND, _N), lambda h, i: (h, 0, 0)),
            pl.BlockSpec((1, 1, 1), lambda h, i: (h, 0, 0)),
        ],
        out_specs=pl.BlockSpec((1, blk2, _N), lambda h, i: (h, i, 0)),
        out_shape=jax.ShapeDtypeStruct((_H, _N, _N), f32),
    )(E1, E2, t_row.reshape(_H, 1, 1))

    dyn_s, final = pl.pallas_call(
        _dyn_body,
        grid=(_B, _N // blk3),
        in_specs=[
            pl.BlockSpec((1, blk3, _ND), lambda b, i: (b, i, 0)),
            pl.BlockSpec((_H, _ND, _N), lambda b, i: (0, 0, 0)),
            pl.BlockSpec((1, _H), lambda b, i: (0, 0)),
            pl.BlockSpec((_H, blk3, _N), lambda b, i: (0, i, 0)),
            pl.BlockSpec((1, 1), lambda b, i: (0, 0)),
            pl.BlockSpec((_H, 2 * _H), lambda b, i: (0, 0)),
            pl.BlockSpec((1, 2 * _H), lambda b, i: (0, 0)),
            pl.BlockSpec((1, 2 * _H), lambda b, i: (0, 0)),
            pl.BlockSpec((1, 1), lambda b, i: (0, 0)),
        ],
        out_specs=[
            pl.BlockSpec((1, _H, blk3, _N), lambda b, i: (b, 0, i, 0)),
            pl.BlockSpec((1, blk3, _N), lambda b, i: (b, i, 0)),
        ],
        out_shape=[
            jax.ShapeDtypeStruct((_B, _H, _N, _N), f32),
            jax.ShapeDtypeStruct((_B, _N, _N), f32),
        ],
    )(dyn_emb, E2, t_row, static_s, fw, we1, be1, we2_row, be2)

    return final, static_s, dyn_s


@jax.jit
def kernel(patch_features, E1, E2, temperature, fusion_weight, W1, b1, ln_g,
           ln_b, W2, b2, We1, be1, We2, be2):
    f32 = jnp.float32
    pf_flat = patch_features.reshape(_B, _N, _P * _D)
    avg = jnp.tile(jnp.eye(_D, dtype=f32), (_P, 1)) * (1.0 / _P)
    t_row = temperature.reshape(1, _H)
    fw = fusion_weight.reshape(1, 1)
    be2r = be2.reshape(1, 1)
    we2_row = We2.reshape(1, 2 * _H)
    final, static_s, dyn_s = _run(
        pf_flat, avg, E1, E2, t_row, W1, b1.reshape(1, _ND),
        ln_g.reshape(1, _ND), ln_b.reshape(1, _ND), W2, b2.reshape(1, _ND),
        fw, We1, be1.reshape(1, 2 * _H), we2_row, be2r)
    return final, static_s, dyn_s
